# baseline (speedup 1.0000x reference)
import functools

import jax
import jax.numpy as jnp
from jax import lax
from jax.experimental import pallas as pl
from jax.experimental.pallas import tpu as pltpu
from jax.experimental.pallas import tpu_sc as plsc

_NUM_ROWS = 16384 * 50
_D = 32
_NC, _NS = 2, 16
_NW = _NC * _NS
_PER_W = _NUM_ROWS // _NW
_STREAM = 256
_N_STREAM = _PER_W // _STREAM
_BLK_STREAMS = 5
_BLK_ROWS = _STREAM * _BLK_STREAMS
_N_BLK = _PER_W // _BLK_ROWS

_mesh = plsc.VectorSubcoreMesh(
    core_axis_name="c", subcore_axis_name="s", num_cores=_NC, num_subcores=_NS
)


@functools.partial(
    pl.kernel,
    out_type=jax.ShapeDtypeStruct((_NUM_ROWS, _D), jnp.float32),
    mesh=_mesh,
    compiler_params=pltpu.CompilerParams(use_tc_tiling_on_sc=False),
    scratch_types=[
        pltpu.VMEM((_N_STREAM, _STREAM), jnp.int32),
        pltpu.VMEM((2, _BLK_ROWS, _D), jnp.float32),
        pltpu.SemaphoreType.DMA,
        pltpu.SemaphoreType.DMA,
        pltpu.SemaphoreType.DMA,
        pltpu.SemaphoreType.DMA,
    ],
)
def _gather_kernel(
    idx_hbm, table_hbm, out_hbm, idx_v, rows_v, gsem0, gsem1, osem0, osem1
):
    wid = lax.axis_index("s") * _NC + lax.axis_index("c")
    base = wid * _PER_W
    gsems = (gsem0, gsem1)
    osems = (osem0, osem1)

    pltpu.sync_copy(idx_hbm.at[wid], idx_v)

    def fire_block(b, buf):
        for j in range(_BLK_STREAMS):
            pltpu.async_copy(
                table_hbm.at[idx_v.at[b * _BLK_STREAMS + j]],
                rows_v.at[buf].at[pl.ds(j * _STREAM, _STREAM)],
                gsems[buf],
            )

    def wait_block(b, buf):
        for j in range(_BLK_STREAMS):
            pltpu.make_async_copy(
                table_hbm.at[idx_v.at[b * _BLK_STREAMS + j]],
                rows_v.at[buf].at[pl.ds(j * _STREAM, _STREAM)],
                gsems[buf],
            ).wait()

    def write_out(b, buf):
        pltpu.async_copy(
            rows_v.at[buf],
            out_hbm.at[pl.ds(base + b * _BLK_ROWS, _BLK_ROWS)],
            osems[buf],
        ).wait()

    fire_block(0, 0)
    fire_block(1, 1)

    def body(i, _):
        for k in range(2):
            b = 2 * i + k
            wait_block(b, k)
            write_out(b, k)
            fire_block(b + 2, k)
        return 0

    lax.fori_loop(0, _N_BLK // 2 - 1, body, 0)

    wait_block(_N_BLK - 2, 0)
    write_out(_N_BLK - 2, 0)
    wait_block(_N_BLK - 1, 1)
    write_out(_N_BLK - 1, 1)


def kernel(input_, weight):
    idx = input_.reshape(_NW, _N_STREAM, _STREAM).astype(jnp.int32)
    out = _gather_kernel(idx, weight)
    return out.reshape(input_.shape[0], input_.shape[1], _D)


# speedup vs baseline: 1.6161x; 1.6161x over previous
import functools

import jax
import jax.numpy as jnp
from jax import lax
from jax.experimental import pallas as pl
from jax.experimental.pallas import tpu as pltpu
from jax.experimental.pallas import tpu_sc as plsc

_B = 16384
_S = 50
_D = 32
_NC, _NS = 2, 16
_NW = _NC * _NS
_B_PER_W = _B // _NW
_BLK_B = 8
_N_BLK = _B_PER_W // _BLK_B

_mesh = plsc.VectorSubcoreMesh(
    core_axis_name="c", subcore_axis_name="s", num_cores=_NC, num_subcores=_NS
)


@functools.partial(
    pl.kernel,
    out_type=jax.ShapeDtypeStruct((_B, _S, _D), jnp.float32),
    mesh=_mesh,
    compiler_params=pltpu.CompilerParams(use_tc_tiling_on_sc=False),
    scratch_types=[
        pltpu.VMEM((_B_PER_W, _S), jnp.int32),
        pltpu.VMEM((2, _BLK_B, _S, _D), jnp.float32),
        pltpu.SemaphoreType.DMA,
        pltpu.SemaphoreType.DMA,
        pltpu.SemaphoreType.DMA,
        pltpu.SemaphoreType.DMA,
    ],
)
def _gather_kernel(
    idx_hbm, table_hbm, out_hbm, idx_v, rows_v, gsem0, gsem1, osem0, osem1
):
    wid = lax.axis_index("s") * _NC + lax.axis_index("c")
    b_base = wid * _B_PER_W
    gsems = (gsem0, gsem1)
    osems = (osem0, osem1)

    pltpu.sync_copy(idx_hbm.at[wid], idx_v)

    def fire_block(blk, buf):
        for j in range(_BLK_B):
            pltpu.async_copy(
                table_hbm.at[idx_v.at[blk * _BLK_B + j]],
                rows_v.at[buf, j],
                gsems[buf],
            )

    def wait_block(blk, buf):
        for j in range(_BLK_B):
            pltpu.make_async_copy(
                table_hbm.at[idx_v.at[blk * _BLK_B + j]],
                rows_v.at[buf, j],
                gsems[buf],
            ).wait()

    def write_out(blk, buf):
        pltpu.async_copy(
            rows_v.at[buf],
            out_hbm.at[pl.ds(b_base + blk * _BLK_B, _BLK_B)],
            osems[buf],
        ).wait()

    fire_block(0, 0)
    fire_block(1, 1)

    def body(i, _):
        for k in range(2):
            blk = 2 * i + k
            wait_block(blk, k)
            write_out(blk, k)
            fire_block(blk + 2, k)
        return 0

    lax.fori_loop(0, _N_BLK // 2 - 1, body, 0)

    wait_block(_N_BLK - 2, 0)
    write_out(_N_BLK - 2, 0)
    wait_block(_N_BLK - 1, 1)
    write_out(_N_BLK - 1, 1)


def kernel(input_, weight):
    idx = input_.reshape(_NW, _B_PER_W, _S).astype(jnp.int32)
    return _gather_kernel(idx, weight)
